# baseline (speedup 1.0000x reference)
import functools

import jax
import jax.numpy as jnp
from jax import lax
from jax.experimental import pallas as pl
from jax.experimental.pallas import tpu as pltpu
from jax.experimental.pallas import tpu_sc as plsc

_NUM_CORES = 2
_NUM_SUBCORES = 16
_NUM_WORKERS = _NUM_CORES * _NUM_SUBCORES
_LANES = 16


@functools.partial(jax.jit, static_argnames=("batch", "seq_len"))
def _embed_sc(ids_flat, wte, wpe_bf16, *, batch, seq_len):
    n_tok = ids_flat.shape[0]
    n_embd = wte.shape[1]
    k = seq_len // _NUM_WORKERS
    groups_per_row = n_embd // (2 * _LANES)

    mesh = plsc.VectorSubcoreMesh(
        core_axis_name="c",
        subcore_axis_name="s",
        num_cores=_NUM_CORES,
        num_subcores=_NUM_SUBCORES,
    )

    @functools.partial(
        pl.kernel,
        out_type=jax.ShapeDtypeStruct((n_tok, n_embd), jnp.float32),
        mesh=mesh,
        scratch_types=[
            pltpu.VMEM((batch * k,), jnp.int32),
            pltpu.VMEM((2, k, n_embd), jnp.float32),
            pltpu.VMEM((k, n_embd // 2), jnp.int32),
            pltpu.SemaphoreType.DMA,
            pltpu.SemaphoreType.DMA,
            pltpu.SemaphoreType.DMA,
            pltpu.SemaphoreType.DMA,
        ],
    )
    def body(ids_hbm, wte_hbm, wpe_hbm, out_hbm, idx_v, rows_v, wpe_v,
             sem_g0, sem_g1, sem_s0, sem_s1):
        wid = lax.axis_index("s") * _NUM_CORES + lax.axis_index("c")
        s0 = wid * k
        sem_g = (sem_g0, sem_g1)
        sem_s = (sem_s0, sem_s1)

        wpe_load = pltpu.async_copy(wpe_hbm.at[pl.ds(s0, k), :], wpe_v, sem_s0)
        id_loads = [
            pltpu.async_copy(
                ids_hbm.at[pl.ds(b * seq_len + s0, k)],
                idx_v.at[pl.ds(b * k, k)],
                sem_g0,
            )
            for b in range(batch)
        ]
        for ld in id_loads:
            ld.wait()

        def start_gather(b):
            return pltpu.async_copy(
                wte_hbm.at[idx_v.at[pl.ds(b * k, k)]], rows_v.at[b % 2],
                sem_g[b % 2],
            )

        gathers = [None] * batch
        stores = [None] * batch
        gathers[0] = start_gather(0)
        if batch > 1:
            gathers[1] = start_gather(1)
        wpe_load.wait()

        for b in range(batch):
            bi = b % 2
            gathers[b].wait()

            def add_row(r):
                for j in range(groups_per_row):
                    w_i32 = wpe_v[r, pl.ds(j * _LANES, _LANES)]
                    lo = lax.bitcast_convert_type(
                        lax.shift_left(w_i32, jnp.int32(16)), jnp.float32
                    )
                    hi = lax.bitcast_convert_type(
                        lax.bitwise_and(w_i32, jnp.int32(-65536)), jnp.float32
                    )
                    rows_v[bi, r, pl.ds(j * 2 * _LANES, _LANES)] += lo
                    rows_v[bi, r, pl.ds(j * 2 * _LANES + _LANES, _LANES)] += hi

            pl.loop(0, k)(add_row)
            stores[b] = pltpu.async_copy(
                rows_v.at[bi], out_hbm.at[pl.ds(b * seq_len + s0, k), :],
                sem_s[bi],
            )
            if b + 2 < batch:
                stores[b].wait()
                gathers[b + 2] = start_gather(b + 2)
        for b in range(max(0, batch - 2), batch):
            stores[b].wait()

    return body(ids_flat, wte, wpe_bf16)


def kernel(input_ids, wte, wpe):
    batch, seq_len = input_ids.shape
    n_embd = wte.shape[1]
    wpe_pairs = wpe.reshape(seq_len, n_embd // 32, 2, 16)
    wpe_inter = jnp.stack(
        [wpe_pairs[:, :, 0, :], wpe_pairs[:, :, 1, :]], axis=-1
    ).reshape(seq_len, n_embd // 2, 2)
    wpe_i32 = lax.bitcast_convert_type(
        wpe_inter.astype(jnp.bfloat16), jnp.int32
    )
    out = _embed_sc(
        input_ids.reshape(-1), wte, wpe_i32, batch=batch, seq_len=seq_len
    )
    return out.reshape(batch, seq_len, n_embd)


# speedup vs baseline: 1.0320x; 1.0320x over previous
import functools

import jax
import jax.numpy as jnp
from jax import lax
from jax.experimental import pallas as pl
from jax.experimental.pallas import tpu as pltpu
from jax.experimental.pallas import tpu_sc as plsc

_NUM_CORES = 2
_NUM_SUBCORES = 16
_NUM_WORKERS = _NUM_CORES * _NUM_SUBCORES
_LANES = 16


@functools.partial(jax.jit, static_argnames=("batch", "seq_len"))
def _embed_sc(ids_flat, wte, wpe, *, batch, seq_len):
    n_tok = ids_flat.shape[0]
    n_embd = wte.shape[1]
    k = seq_len // _NUM_WORKERS
    h = k // 2
    lanes_per_row = n_embd // _LANES

    mesh = plsc.VectorSubcoreMesh(
        core_axis_name="c",
        subcore_axis_name="s",
        num_cores=_NUM_CORES,
        num_subcores=_NUM_SUBCORES,
    )

    @functools.partial(
        pl.kernel,
        out_type=jax.ShapeDtypeStruct((n_tok, n_embd), jnp.float32),
        mesh=mesh,
        scratch_types=[
            pltpu.VMEM((batch * k,), jnp.int32),
            pltpu.VMEM((k, n_embd), jnp.float32),
            pltpu.VMEM((k, n_embd), jnp.float32),
            pltpu.SemaphoreType.DMA,
            pltpu.SemaphoreType.DMA,
            pltpu.SemaphoreType.DMA,
            pltpu.SemaphoreType.DMA,
        ],
    )
    def body(ids_hbm, wte_hbm, wpe_hbm, out_hbm, idx_v, rows_v, wpe_v,
             sem_ga, sem_gb, sem_sa, sem_sb):
        wid = lax.axis_index("s") * _NUM_CORES + lax.axis_index("c")
        s0 = wid * k

        wpe_load = pltpu.async_copy(wpe_hbm.at[pl.ds(s0, k), :], wpe_v, sem_sa)
        id_loads = [
            pltpu.async_copy(
                ids_hbm.at[pl.ds(b * seq_len + s0, k)],
                idx_v.at[pl.ds(b * k, k)],
                sem_ga,
            )
            for b in range(batch)
        ]
        for ld in id_loads:
            ld.wait()

        def start_gather(b, half, sem):
            return pltpu.async_copy(
                wte_hbm.at[idx_v.at[pl.ds(b * k + half * h, h)]],
                rows_v.at[pl.ds(half * h, h)],
                sem,
            )

        def add_half(half):
            def add_row(r):
                for j in range(lanes_per_row):
                    sl = pl.ds(j * _LANES, _LANES)
                    rows_v[half * h + r, sl] += wpe_v[half * h + r, sl]

            pl.loop(0, h)(add_row)

        ga = start_gather(0, 0, sem_ga)
        gb = start_gather(0, 1, sem_gb)
        wpe_load.wait()

        sa = sb = None
        for b in range(batch):
            base = b * seq_len + s0
            ga.wait()
            add_half(0)
            sa_new = pltpu.async_copy(
                rows_v.at[pl.ds(0, h)], out_hbm.at[pl.ds(base, h), :], sem_sa
            )
            gb.wait()
            add_half(1)
            sb_new = pltpu.async_copy(
                rows_v.at[pl.ds(h, h)], out_hbm.at[pl.ds(base + h, h), :], sem_sb
            )
            sa, sb = sa_new, sb_new
            if b + 1 < batch:
                sa.wait()
                ga = start_gather(b + 1, 0, sem_ga)
                sb.wait()
                gb = start_gather(b + 1, 1, sem_gb)
        sa.wait()
        sb.wait()

    return body(ids_flat, wte, wpe)


def kernel(input_ids, wte, wpe):
    batch, seq_len = input_ids.shape
    out = _embed_sc(input_ids.reshape(-1), wte, wpe, batch=batch, seq_len=seq_len)
    return out.reshape(batch, seq_len, wte.shape[1])


# speedup vs baseline: 1.2123x; 1.1747x over previous
import functools

import jax
import jax.numpy as jnp
from jax import lax
from jax.experimental import pallas as pl
from jax.experimental.pallas import tpu as pltpu
from jax.experimental.pallas import tpu_sc as plsc

_NUM_CORES = 2
_NUM_SUBCORES = 16
_NUM_WORKERS = _NUM_CORES * _NUM_SUBCORES
_LANES = 16


@functools.partial(jax.jit, static_argnames=("batch", "seq_len"))
def _embed_sc(ids_flat, wte, wpe, *, batch, seq_len):
    n_tok = ids_flat.shape[0]
    n_embd = wte.shape[1]
    k = seq_len // _NUM_WORKERS
    lanes_per_row = n_embd // _LANES

    mesh = plsc.VectorSubcoreMesh(
        core_axis_name="c",
        subcore_axis_name="s",
        num_cores=_NUM_CORES,
        num_subcores=_NUM_SUBCORES,
    )

    @functools.partial(
        pl.kernel,
        out_type=jax.ShapeDtypeStruct((n_tok, n_embd), jnp.float32),
        mesh=mesh,
        scratch_types=[
            pltpu.VMEM((batch * k,), jnp.int32),
            pltpu.VMEM((k, n_embd), jnp.float32),
            pltpu.VMEM((k, n_embd), jnp.float32),
            pltpu.SemaphoreType.DMA,
            pltpu.SemaphoreType.DMA,
        ],
    )
    def body(ids_hbm, wte_hbm, wpe_hbm, out_hbm, idx_v, rows_v, wpe_v,
             sem_g, sem_p):
        wid = lax.axis_index("s") * _NUM_CORES + lax.axis_index("c")
        s0 = wid * k

        wpe_load = pltpu.async_copy(wpe_hbm.at[pl.ds(s0, k), :], wpe_v, sem_p)
        id_loads = [
            pltpu.async_copy(
                ids_hbm.at[pl.ds(b * seq_len + s0, k)],
                idx_v.at[pl.ds(b * k, k)],
                sem_g,
            )
            for b in range(batch)
        ]
        for ld in id_loads:
            ld.wait()
        wpe_load.wait()

        for b in range(batch):
            base = b * seq_len + s0
            gather = pltpu.async_copy(
                wte_hbm.at[idx_v.at[pl.ds(b * k, k)]], rows_v, sem_g
            )
            gather.wait()

            def add_row(r):
                for j in range(lanes_per_row):
                    sl = pl.ds(j * _LANES, _LANES)
                    rows_v[r, sl] += wpe_v[r, sl]

            plsc.parallel_loop(0, k, 1, unroll=4)(add_row)
            pltpu.sync_copy(rows_v, out_hbm.at[pl.ds(base, k), :])

    return body(ids_flat, wte, wpe)


def kernel(input_ids, wte, wpe):
    batch, seq_len = input_ids.shape
    out = _embed_sc(input_ids.reshape(-1), wte, wpe, batch=batch, seq_len=seq_len)
    return out.reshape(batch, seq_len, wte.shape[1])


# speedup vs baseline: 1.3094x; 1.0801x over previous
import functools

import jax
import jax.numpy as jnp
from jax import lax
from jax.experimental import pallas as pl
from jax.experimental.pallas import tpu as pltpu
from jax.experimental.pallas import tpu_sc as plsc

_NUM_CORES = 2
_NUM_SUBCORES = 16
_NUM_WORKERS = _NUM_CORES * _NUM_SUBCORES
_LANES = 16


@functools.partial(jax.jit, static_argnames=("batch", "seq_len"))
def _embed_sc(ids_flat, wte, wpe, *, batch, seq_len):
    n_tok = ids_flat.shape[0]
    n_embd = wte.shape[1]
    k = seq_len // _NUM_WORKERS
    lanes_per_row = n_embd // _LANES

    mesh = plsc.VectorSubcoreMesh(
        core_axis_name="c",
        subcore_axis_name="s",
        num_cores=_NUM_CORES,
        num_subcores=_NUM_SUBCORES,
    )

    @functools.partial(
        pl.kernel,
        out_type=jax.ShapeDtypeStruct((n_tok, n_embd), jnp.float32),
        mesh=mesh,
        scratch_types=[
            pltpu.VMEM((batch * k,), jnp.int32),
            pltpu.VMEM((k, n_embd), jnp.float32),
            pltpu.VMEM((k, n_embd), jnp.float32),
            pltpu.SemaphoreType.DMA,
            pltpu.SemaphoreType.DMA,
        ],
    )
    def body(ids_hbm, wte_hbm, wpe_hbm, out_hbm, idx_v, rows_v, wpe_v,
             sem_g, sem_p):
        wid = lax.axis_index("s") * _NUM_CORES + lax.axis_index("c")
        s0 = wid * k

        wpe_load = pltpu.async_copy(wpe_hbm.at[pl.ds(s0, k), :], wpe_v, sem_p)
        id_loads = [
            pltpu.async_copy(
                ids_hbm.at[pl.ds(b * seq_len + s0, k)],
                idx_v.at[pl.ds(b * k, k)],
                sem_g,
            )
            for b in range(batch)
        ]
        for ld in id_loads:
            ld.wait()
        wpe_load.wait()

        for b in range(batch):
            base = b * seq_len + s0
            gather = pltpu.async_copy(
                wte_hbm.at[idx_v.at[pl.ds(b * k, k)]], rows_v, sem_g
            )
            gather.wait()

            def add_row(r):
                for j in range(lanes_per_row):
                    sl = pl.ds(j * _LANES, _LANES)
                    plsc.addupdate(rows_v.at[r, sl], wpe_v[r, sl])

            pl.loop(0, k)(add_row)
            pltpu.sync_copy(rows_v, out_hbm.at[pl.ds(base, k), :])

    return body(ids_flat, wte, wpe)


def kernel(input_ids, wte, wpe):
    batch, seq_len = input_ids.shape
    out = _embed_sc(input_ids.reshape(-1), wte, wpe, batch=batch, seq_len=seq_len)
    return out.reshape(batch, seq_len, wte.shape[1])


# speedup vs baseline: 1.3466x; 1.0284x over previous
import functools

import jax
import jax.numpy as jnp
from jax import lax
from jax.experimental import pallas as pl
from jax.experimental.pallas import tpu as pltpu
from jax.experimental.pallas import tpu_sc as plsc

_NUM_CORES = 2
_NUM_SUBCORES = 16
_NUM_WORKERS = _NUM_CORES * _NUM_SUBCORES
_LANES = 16


@functools.partial(jax.jit, static_argnames=("batch", "seq_len"))
def _embed_sc(ids_flat, wte, wpe, *, batch, seq_len):
    n_embd = wte.shape[1]
    k = seq_len // _NUM_WORKERS
    lanes_per_row = n_embd // _LANES

    mesh = plsc.VectorSubcoreMesh(
        core_axis_name="c",
        subcore_axis_name="s",
        num_cores=_NUM_CORES,
        num_subcores=_NUM_SUBCORES,
    )

    @functools.partial(
        pl.kernel,
        out_type=jax.ShapeDtypeStruct((batch * seq_len, n_embd), jnp.float32),
        mesh=mesh,
        scratch_types=[
            pltpu.VMEM((batch * k,), jnp.int32),
            pltpu.VMEM((k, n_embd), jnp.float32),
            pltpu.VMEM((k, n_embd), jnp.float32),
            pltpu.SemaphoreType.DMA,
            pltpu.SemaphoreType.DMA,
        ],
    )
    def body(ids_hbm, wte_hbm, wpe_hbm, out_hbm, idx_v, rows_v, wpe_v,
             sem_g, sem_p):
        wid = lax.axis_index("s") * _NUM_CORES + lax.axis_index("c")
        s0 = wid * k

        wpe_load = pltpu.async_copy(wpe_hbm.at[pl.ds(s0, k), :], wpe_v, sem_p)
        id_loads = [
            pltpu.async_copy(
                ids_hbm.at[pl.ds(b * seq_len + s0, k)],
                idx_v.at[pl.ds(b * k, k)],
                sem_g,
            )
            for b in range(batch)
        ]
        id_loads[0].wait()
        gather = pltpu.async_copy(
            wte_hbm.at[idx_v.at[pl.ds(0, k)]], rows_v, sem_g
        )
        for ld in id_loads[1:]:
            ld.wait()
        wpe_load.wait()

        for b in range(batch):
            base = b * seq_len + s0
            gather.wait()

            def add_row(r):
                for j in range(lanes_per_row):
                    sl = pl.ds(j * _LANES, _LANES)
                    rows_v[r, sl] += wpe_v[r, sl]

            pl.loop(0, k)(add_row)
            pltpu.sync_copy(rows_v, out_hbm.at[pl.ds(base, k), :])
            if b + 1 < batch:
                gather = pltpu.async_copy(
                    wte_hbm.at[idx_v.at[pl.ds((b + 1) * k, k)]], rows_v, sem_g
                )

    return body(ids_flat, wte, wpe)


def kernel(input_ids, wte, wpe):
    batch, seq_len = input_ids.shape
    out = _embed_sc(input_ids.reshape(-1), wte, wpe, batch=batch, seq_len=seq_len)
    return out.reshape(batch, seq_len, wte.shape[1])
